# baseline (speedup 1.0000x reference)
import functools

import jax
import jax.numpy as jnp
from jax import lax
from jax.experimental import pallas as pl
from jax.experimental.pallas import tpu as pltpu
from jax.experimental.pallas import tpu_sc as plsc

_VOCAB = 100000
_EMBED = 32
_BATCH = 1024

_BB = 32
_VC = 2048
_NCHUNK = 49
_VPAD = _NCHUNK * _VC


def _sc_gather(table, idx):
    info = plsc.get_sparse_core_info()
    nw = info.num_cores * info.num_subcores
    b_per_w = _BATCH // nw
    mesh = plsc.VectorSubcoreMesh(core_axis_name="c", subcore_axis_name="s")

    @functools.partial(
        pl.kernel,
        out_type=jax.ShapeDtypeStruct((_BATCH, _EMBED), jnp.float32),
        mesh=mesh,
        scratch_types=[
            pltpu.VMEM((b_per_w,), jnp.int32),
            pltpu.VMEM((b_per_w, _EMBED), jnp.float32),
            pltpu.SemaphoreType.DMA,
        ],
        compiler_params=pltpu.CompilerParams(use_tc_tiling_on_sc=False),
    )
    def gather_kernel(table_hbm, idx_hbm, out_hbm, idx_v, rows_v, sem):
        wid = lax.axis_index("s") * info.num_cores + lax.axis_index("c")
        base = wid * b_per_w
        pltpu.sync_copy(idx_hbm.at[pl.ds(base, b_per_w)], idx_v)
        pltpu.async_copy(table_hbm.at[idx_v], rows_v, sem).wait()
        pltpu.sync_copy(rows_v, out_hbm.at[pl.ds(base, b_per_w)])

    return gather_kernel(table, idx)


def _tc_softmax_body(emb_ref, wt_ref, out_ref):
    emb = emb_ref[...]

    m = jnp.full((_BB, 1), -jnp.inf, dtype=jnp.float32)
    for j in range(_NCHUNK):
        ch = jnp.dot(
            emb, wt_ref[:, j * _VC:(j + 1) * _VC],
            preferred_element_type=jnp.float32,
        )
        if j == _NCHUNK - 1:
            col = j * _VC + lax.broadcasted_iota(jnp.int32, (_BB, _VC), 1)
            ch = jnp.where(col < _VOCAB, ch, -jnp.inf)
        out_ref[:, j * _VC:(j + 1) * _VC] = ch
        m = jnp.maximum(m, jnp.max(ch, axis=1, keepdims=True))

    s = jnp.zeros((_BB, 1), dtype=jnp.float32)
    for j in range(_NCHUNK):
        e = jnp.exp(out_ref[:, j * _VC:(j + 1) * _VC] - m)
        out_ref[:, j * _VC:(j + 1) * _VC] = e
        s = s + jnp.sum(e, axis=1, keepdims=True)

    r = 1.0 / s
    for j in range(_NCHUNK):
        out_ref[:, j * _VC:(j + 1) * _VC] = out_ref[:, j * _VC:(j + 1) * _VC] * r


def _tc_softmax(emb_sel, wt):
    return pl.pallas_call(
        _tc_softmax_body,
        grid=(_BATCH // _BB,),
        in_specs=[
            pl.BlockSpec((_BB, _EMBED), lambda b: (b, 0)),
            pl.BlockSpec((_EMBED, _VPAD), lambda b: (0, 0)),
        ],
        out_specs=pl.BlockSpec((_BB, _VPAD), lambda b: (b, 0)),
        out_shape=jax.ShapeDtypeStruct((_BATCH, _VOCAB), jnp.float32),
    )(emb_sel, wt)


def kernel(inputs, emb_table, W):
    emb_sel = _sc_gather(emb_table, inputs.astype(jnp.int32))
    wt = W.T
    return _tc_softmax(emb_sel, wt)
